# baseline (speedup 1.0000x reference)
import functools

import jax
import jax.numpy as jnp
from jax import lax
from jax.experimental import pallas as pl
from jax.experimental.pallas import tpu as pltpu
from jax.experimental.pallas import tpu_sc as plsc

NUM_CORES = 2
NUM_SUBCORES = 16
NW = NUM_CORES * NUM_SUBCORES
BATCH = 16384
EMBED = 64
HIDDEN = 128
NROW = 1000000
BPW = BATCH // NW
EPW = BPW * EMBED
CHUNK = 128
NCH = EPW // CHUNK


@functools.lru_cache(maxsize=1)
def _get_sc_gather():
    mesh = plsc.VectorSubcoreMesh(
        core_axis_name="c", subcore_axis_name="s",
        num_cores=NUM_CORES, num_subcores=NUM_SUBCORES)

    @functools.partial(
        pl.kernel,
        out_type=jax.ShapeDtypeStruct((2 * EMBED, BATCH), jnp.float32),
        mesh=mesh,
        scratch_types=(
            pltpu.VMEM((BPW,), jnp.int32),
            pltpu.VMEM((NCH, CHUNK), jnp.int32),
            pltpu.VMEM((EMBED, BPW), jnp.float32),
            pltpu.SemaphoreType.DMA,
        ),
        compiler_params=pltpu.CompilerParams(use_tc_tiling_on_sc=False),
    )
    def sc_gather(uid_hbm, vid_hbm, ut_hbm, vt_hbm, xt_out,
                  ids, idx, rows, sem):
        wid = lax.axis_index("s") * NUM_CORES + lax.axis_index("c")
        base = wid * BPW

        def one_table(id_hbm, tab_hbm, out_row0):
            pltpu.sync_copy(id_hbm.at[pl.ds(base, BPW)], ids)

            def build(c, _):
                a = c * NROW
                for g in range(BPW // 16):
                    row = c * 4 + (g // 8)
                    col = (g % 8) * 16
                    idx[row, pl.ds(col, 16)] = ids[pl.ds(g * 16, 16)] + a
                return 0

            lax.fori_loop(0, EMBED, build, 0)

            def fire(j, _):
                pltpu.async_copy(tab_hbm.at[idx.at[j]],
                                 rows.at[j // 4, pl.ds((j % 4) * CHUNK, CHUNK)],
                                 sem)
                return 0

            lax.fori_loop(0, NCH, fire, 0)

            def drain(j, _):
                pltpu.make_async_copy(
                    tab_hbm.at[idx.at[j]],
                    rows.at[j // 4, pl.ds((j % 4) * CHUNK, CHUNK)],
                    sem).wait()
                return 0

            lax.fori_loop(0, NCH, drain, 0)
            pltpu.sync_copy(rows, xt_out.at[pl.ds(out_row0, EMBED),
                                            pl.ds(base, BPW)])

        one_table(uid_hbm, ut_hbm, 0)
        one_table(vid_hbm, vt_hbm, EMBED)

    return sc_gather


BM = 2048


def _mlp_body(xt_ref, w1_ref, b1_ref, w2_ref, b2_ref, out_ref):
    h = lax.dot_general(w1_ref[...], xt_ref[...], (((0,), (0,)), ((), ())),
                        preferred_element_type=jnp.float32)
    h = jnp.maximum(h + b1_ref[...], 0.0)
    o = lax.dot_general(h, w2_ref[...], (((0,), (0,)), ((), ())),
                        preferred_element_type=jnp.float32)
    out_ref[...] = jax.nn.sigmoid(o + b2_ref[0, 0])


_mlp = pl.pallas_call(
    _mlp_body,
    grid=(BATCH // BM,),
    in_specs=[
        pl.BlockSpec((2 * EMBED, BM), lambda i: (0, i)),
        pl.BlockSpec((2 * EMBED, HIDDEN), lambda i: (0, 0)),
        pl.BlockSpec((HIDDEN, 1), lambda i: (0, 0)),
        pl.BlockSpec((HIDDEN, 1), lambda i: (0, 0)),
        pl.BlockSpec((1, 1), lambda i: (0, 0)),
    ],
    out_specs=pl.BlockSpec((BM, 1), lambda i: (i, 0)),
    out_shape=jax.ShapeDtypeStruct((BATCH, 1), jnp.float32),
)


def kernel(user_id, video_id, user_table, video_table, W1, b1, W2, b2):
    uflat = user_table.T.reshape(-1)
    vflat = video_table.T.reshape(-1)
    xt = _get_sc_gather()(user_id, video_id, uflat, vflat)
    return _mlp(xt, W1, b1.reshape(HIDDEN, 1), W2, b2.reshape(1, 1))


# speedup vs baseline: 12.1314x; 12.1314x over previous
import functools

import jax
import jax.numpy as jnp
from jax import lax
from jax.experimental import pallas as pl
from jax.experimental.pallas import tpu as pltpu
from jax.experimental.pallas import tpu_sc as plsc

NUM_CORES = 2
NUM_SUBCORES = 16
NW = NUM_CORES * NUM_SUBCORES
BATCH = 16384
EMBED = 64
HIDDEN = 128
NROW = 1000000
CSTRIDE = 1 << 20
FLAT = EMBED * CSTRIDE
DET_BK = 131072
DET_M = CSTRIDE // DET_BK
BPW = BATCH // NW
EPW = BPW * EMBED
CHUNK = 128
NCH = EPW // CHUNK


def _flatten_table(table):
    padded = jnp.pad(table, ((0, CSTRIDE - NROW), (0, 0)))
    return padded.T.reshape(-1)


@functools.lru_cache(maxsize=1)
def _get_sc_gather():
    mesh = plsc.VectorSubcoreMesh(
        core_axis_name="c", subcore_axis_name="s",
        num_cores=NUM_CORES, num_subcores=NUM_SUBCORES)

    @functools.partial(
        pl.kernel,
        out_type=jax.ShapeDtypeStruct((2 * EMBED, BATCH), jnp.float32),
        mesh=mesh,
        scratch_types=(
            pltpu.VMEM((BPW,), jnp.int32),
            pltpu.VMEM((NCH, CHUNK), jnp.int32),
            pltpu.VMEM((EMBED, BPW), jnp.float32),
            pltpu.SemaphoreType.DMA,
        ),
        compiler_params=pltpu.CompilerParams(use_tc_tiling_on_sc=False),
    )
    def sc_gather(uid_hbm, vid_hbm, ut_hbm, vt_hbm, xt_out,
                  ids, idx, rows, sem):
        wid = lax.axis_index("s") * NUM_CORES + lax.axis_index("c")
        base = wid * BPW

        def one_table(id_hbm, tab_hbm, out_row0):
            pltpu.sync_copy(id_hbm.at[pl.ds(base, BPW)], ids)

            def build(c, _):
                a = c * CSTRIDE
                for g in range(BPW // 16):
                    row = c * 4 + (g // 8)
                    col = (g % 8) * 16
                    idx[row, pl.ds(col, 16)] = ids[pl.ds(g * 16, 16)] + a
                return 0

            lax.fori_loop(0, EMBED, build, 0)

            def fire(j, _):
                pltpu.async_copy(tab_hbm.at[idx.at[j]],
                                 rows.at[j // 4, pl.ds((j % 4) * CHUNK, CHUNK)],
                                 sem)
                return 0

            lax.fori_loop(0, NCH, fire, 0)

            def drain(j, _):
                pltpu.make_async_copy(
                    tab_hbm.at[idx.at[j]],
                    rows.at[j // 4, pl.ds((j % 4) * CHUNK, CHUNK)],
                    sem).wait()
                return 0

            lax.fori_loop(0, NCH, drain, 0)
            pltpu.sync_copy(rows, xt_out.at[pl.ds(out_row0, EMBED),
                                            pl.ds(base, BPW)])

        one_table(uid_hbm, ut_hbm, 0)
        one_table(vid_hbm, vt_hbm, EMBED)

    return sc_gather


BM = 2048


def _mlp_body(xt_ref, w1_ref, b1_ref, w2_ref, b2_ref, out_ref):
    h = lax.dot_general(w1_ref[...], xt_ref[...], (((0,), (0,)), ((), ())),
                        preferred_element_type=jnp.float32)
    h = jnp.maximum(h + b1_ref[...], 0.0)
    o = lax.dot_general(h, w2_ref[...], (((0,), (0,)), ((), ())),
                        preferred_element_type=jnp.float32)
    out_ref[...] = jax.nn.sigmoid(o + b2_ref[0, 0])


_mlp = pl.pallas_call(
    _mlp_body,
    grid=(BATCH // BM,),
    in_specs=[
        pl.BlockSpec((2 * EMBED, BM), lambda i: (0, i)),
        pl.BlockSpec((2 * EMBED, HIDDEN), lambda i: (0, 0)),
        pl.BlockSpec((HIDDEN, 1), lambda i: (0, 0)),
        pl.BlockSpec((HIDDEN, 1), lambda i: (0, 0)),
        pl.BlockSpec((1, 1), lambda i: (0, 0)),
    ],
    out_specs=pl.BlockSpec((BM, 1), lambda i: (i, 0)),
    out_shape=jax.ShapeDtypeStruct((BATCH, 1), jnp.float32),
)


def kernel(user_id, video_id, user_table, video_table, W1, b1, W2, b2):
    uflat = _flatten_table(user_table)
    vflat = _flatten_table(video_table)
    xt = _get_sc_gather()(user_id, video_id, uflat, vflat)
    return _mlp(xt, W1, b1.reshape(HIDDEN, 1), W2, b2.reshape(1, 1))


# speedup vs baseline: 19.9705x; 1.6462x over previous
import functools

import jax
import jax.numpy as jnp
from jax import lax
from jax.experimental import pallas as pl
from jax.experimental.pallas import tpu as pltpu
from jax.experimental.pallas import tpu_sc as plsc

NUM_CORES = 2
NUM_SUBCORES = 16
NW = NUM_CORES * NUM_SUBCORES
BATCH = 16384
EMBED = 64
HIDDEN = 128
NROW = 1000000
BPW = BATCH // NW
NRING = 2


@functools.lru_cache(maxsize=1)
def _get_sc_gather():
    mesh = plsc.VectorSubcoreMesh(
        core_axis_name="c", subcore_axis_name="s",
        num_cores=NUM_CORES, num_subcores=NUM_SUBCORES)

    @functools.partial(
        pl.kernel,
        out_type=(jax.ShapeDtypeStruct((BATCH // 2, 2 * EMBED), jnp.float32),
                  jax.ShapeDtypeStruct((BATCH // 2, 2 * EMBED), jnp.float32)),
        mesh=mesh,
        scratch_types=(
            pltpu.VMEM((BPW,), jnp.int32),
            pltpu.VMEM((BPW,), jnp.int32),
            pltpu.VMEM((NRING, EMBED, 128), jnp.float32),
            pltpu.VMEM((NRING, EMBED, 128), jnp.float32),
            pltpu.VMEM((BPW // 2, 2 * EMBED), jnp.float32),
            pltpu.VMEM((BPW // 2, 2 * EMBED), jnp.float32),
            pltpu.SemaphoreType.DMA,
            pltpu.SemaphoreType.DMA,
        ),
        compiler_params=pltpu.CompilerParams(use_tc_tiling_on_sc=True,
                                             needs_layout_passes=False),
    )
    def sc_gather(uid_hbm, vid_hbm, ut_hbm, vt_hbm, u_out, v_out,
                  uids, vids, uring, vring, du, dv, su, sv):
        wid = lax.axis_index("s") * NUM_CORES + lax.axis_index("c")
        base = wid * BPW

        pltpu.sync_copy(uid_hbm.at[pl.ds(base, BPW)], uids)
        pltpu.sync_copy(vid_hbm.at[pl.ds(base, BPW)], vids)

        def id_at(ids, i):
            grp = ids[pl.ds((i // 16) * 16, 16)]
            sel = lax.iota(jnp.int32, 16) == (i % 16)
            return jnp.max(jnp.where(sel, grp, 0))

        def slab_copy(tab, ids, ring, i, sem):
            off = pl.multiple_of((id_at(ids, i) >> 7) * 128, 128)
            return pltpu.make_async_copy(
                tab.at[pl.ds(0, EMBED), pl.ds(off, 128)],
                ring.at[i % NRING], sem)

        slab_copy(ut_hbm, uids, uring, 0, su).start()
        slab_copy(vt_hbm, vids, vring, 0, sv).start()

        def extract(ids, ring, dst, i):
            r = id_at(ids, i)
            lane = jnp.full((16,), r & 127, jnp.int32)
            row = i // 2
            colbase = (i % 2) * EMBED
            for g in range(EMBED // 16):
                cv = lax.iota(jnp.int32, 16) + (g * 16)
                vals = plsc.load_gather(ring.at[i % NRING], [cv, lane])
                dst[row, pl.ds(colbase + g * 16, 16)] = vals

        def step(i, _):
            @pl.when(i < BPW - 1)
            def _prefetch():
                slab_copy(ut_hbm, uids, uring, i + 1, su).start()
                slab_copy(vt_hbm, vids, vring, i + 1, sv).start()

            slab_copy(ut_hbm, uids, uring, i, su).wait()
            extract(uids, uring, du, i)
            slab_copy(vt_hbm, vids, vring, i, sv).wait()
            extract(vids, vring, dv, i)
            return 0

        lax.fori_loop(0, BPW, step, 0)

        pltpu.sync_copy(du, u_out.at[pl.ds(wid * (BPW // 2), BPW // 2)])
        pltpu.sync_copy(dv, v_out.at[pl.ds(wid * (BPW // 2), BPW // 2)])

    return sc_gather


BM = 2048


def _mlp_body(up_ref, vp_ref, we_u, we_v, wo_u, wo_v,
              b1_ref, w2_ref, b2_ref, oe_ref, oo_ref):
    up = up_ref[...]
    vp = vp_ref[...]
    he = jnp.dot(up, we_u[...], preferred_element_type=jnp.float32)
    he = he + jnp.dot(vp, we_v[...], preferred_element_type=jnp.float32)
    he = jnp.maximum(he + b1_ref[...], 0.0)
    oe = jnp.dot(he, w2_ref[...], preferred_element_type=jnp.float32)
    oe_ref[...] = jax.nn.sigmoid(oe + b2_ref[0, 0])
    ho = jnp.dot(up, wo_u[...], preferred_element_type=jnp.float32)
    ho = ho + jnp.dot(vp, wo_v[...], preferred_element_type=jnp.float32)
    ho = jnp.maximum(ho + b1_ref[...], 0.0)
    oo = jnp.dot(ho, w2_ref[...], preferred_element_type=jnp.float32)
    oo_ref[...] = jax.nn.sigmoid(oo + b2_ref[0, 0])


_mlp = pl.pallas_call(
    _mlp_body,
    grid=(BATCH // BM,),
    in_specs=[
        pl.BlockSpec((BM // 2, 2 * EMBED), lambda i: (i, 0)),
        pl.BlockSpec((BM // 2, 2 * EMBED), lambda i: (i, 0)),
        pl.BlockSpec((2 * EMBED, HIDDEN), lambda i: (0, 0)),
        pl.BlockSpec((2 * EMBED, HIDDEN), lambda i: (0, 0)),
        pl.BlockSpec((2 * EMBED, HIDDEN), lambda i: (0, 0)),
        pl.BlockSpec((2 * EMBED, HIDDEN), lambda i: (0, 0)),
        pl.BlockSpec((1, HIDDEN), lambda i: (0, 0)),
        pl.BlockSpec((HIDDEN, 1), lambda i: (0, 0)),
        pl.BlockSpec((1, 1), lambda i: (0, 0)),
    ],
    out_specs=(pl.BlockSpec((BM // 2, 1), lambda i: (i, 0)),
               pl.BlockSpec((BM // 2, 1), lambda i: (i, 0))),
    out_shape=(jax.ShapeDtypeStruct((BATCH // 2, 1), jnp.float32),
               jax.ShapeDtypeStruct((BATCH // 2, 1), jnp.float32)),
)


def kernel(user_id, video_id, user_table, video_table, W1, b1, W2, b2):
    up, vp = _get_sc_gather()(user_id, video_id, user_table.T, video_table.T)
    z = jnp.zeros((EMBED, HIDDEN), jnp.float32)
    we_u = jnp.concatenate([W1[:EMBED], z], axis=0)
    we_v = jnp.concatenate([W1[EMBED:], z], axis=0)
    wo_u = jnp.concatenate([z, W1[:EMBED]], axis=0)
    wo_v = jnp.concatenate([z, W1[EMBED:]], axis=0)
    oe, oo = _mlp(up, vp, we_u, we_v, wo_u, wo_v,
                  b1.reshape(1, HIDDEN), W2, b2.reshape(1, 1))
    return jnp.concatenate([oe, oo], axis=1).reshape(BATCH, 1)
